# baseline (speedup 1.0000x reference)
import functools

import jax
import jax.numpy as jnp
from jax import lax
from jax.experimental import pallas as pl
from jax.experimental.pallas import tpu as pltpu
from jax.experimental.pallas import tpu_sc as plsc

_MAX_SIZE = 1000000
_STATE_DIM = 64
_BATCH = 16384

_NC = 2
_NS = 16
_NW = _NC * _NS
_B_PER_W = _BATCH // _NW
_CHUNK = 128
_NCHUNK = _B_PER_W // _CHUNK


def _make_sample_kernel():
    mesh = plsc.VectorSubcoreMesh(core_axis_name="c", subcore_axis_name="s")

    @functools.partial(
        pl.kernel,
        mesh=mesh,
        compiler_params=pltpu.CompilerParams(use_tc_tiling_on_sc=False),
        out_type=(
            jax.ShapeDtypeStruct((_BATCH, _STATE_DIM), jnp.float32),
            jax.ShapeDtypeStruct((_BATCH,), jnp.int32),
            jax.ShapeDtypeStruct((_BATCH,), jnp.float32),
            jax.ShapeDtypeStruct((_BATCH, _STATE_DIM), jnp.float32),
            jax.ShapeDtypeStruct((_BATCH,), jnp.float32),
        ),
        scratch_types=[
            pltpu.VMEM((_NCHUNK, _CHUNK), jnp.int32),
            pltpu.VMEM((_B_PER_W, _STATE_DIM), jnp.float32),
            pltpu.VMEM((_B_PER_W,), jnp.int32),
            pltpu.VMEM((_B_PER_W,), jnp.float32),
            pltpu.VMEM((_B_PER_W, _STATE_DIM), jnp.float32),
            pltpu.VMEM((_B_PER_W,), jnp.float32),
            pltpu.SemaphoreType.DMA,
        ],
    )
    def sample(s_hbm, a_hbm, r_hbm, sn_hbm, dw_hbm, ind_hbm,
               s_out, a_out, r_out, sn_out, dw_out,
               idx_v, s_v, a_v, r_v, sn_v, dw_v, sem):
        wid = lax.axis_index("s") * _NC + lax.axis_index("c")
        base = wid * _B_PER_W
        pltpu.sync_copy(ind_hbm.at[wid], idx_v)
        copies = []
        for j in range(_NCHUNK):
            idx_j = idx_v.at[j]
            sl = pl.ds(j * _CHUNK, _CHUNK)
            copies.append(pltpu.async_copy(s_hbm.at[idx_j], s_v.at[sl], sem))
            copies.append(pltpu.async_copy(a_hbm.at[idx_j], a_v.at[sl], sem))
            copies.append(pltpu.async_copy(r_hbm.at[idx_j], r_v.at[sl], sem))
            copies.append(pltpu.async_copy(sn_hbm.at[idx_j], sn_v.at[sl], sem))
            copies.append(pltpu.async_copy(dw_hbm.at[idx_j], dw_v.at[sl], sem))
        for c in copies:
            c.wait()
        osl = pl.ds(base, _B_PER_W)
        pltpu.sync_copy(s_v, s_out.at[osl])
        pltpu.sync_copy(a_v, a_out.at[osl])
        pltpu.sync_copy(r_v, r_out.at[osl])
        pltpu.sync_copy(sn_v, sn_out.at[osl])
        pltpu.sync_copy(dw_v, dw_out.at[osl])

    return sample


_sample = _make_sample_kernel()


def kernel(s, a, r, s_next, dw, ind):
    ind3 = ind.reshape(_NW, _NCHUNK, _CHUNK)
    s_b, a_b, r_b, sn_b, dw_b = _sample(
        s, a[:, 0], r[:, 0], s_next, dw[:, 0], ind3)
    return (s_b, a_b.reshape(_BATCH, 1), r_b.reshape(_BATCH, 1),
            sn_b, dw_b.reshape(_BATCH, 1))
